# baseline (speedup 1.0000x reference)
import functools

import jax
import jax.numpy as jnp
from jax import lax
from jax.experimental import pallas as pl
from jax.experimental.pallas import tpu as pltpu
from jax.experimental.pallas import tpu_sc as plsc

D_MODEL = 64
LANES = 16
NUM_CORES = 2
NUM_SUBCORES = 16
NW = NUM_CORES * NUM_SUBCORES

CHUNK = 128
K = 4
ROWS = K * CHUNK
SCALE = 8.0


def _make_lookup(batch_total):
    assert batch_total % (NW * ROWS) == 0
    rows_per_w = batch_total // NW
    nch = rows_per_w // ROWS
    n_idx_rows = rows_per_w // CHUNK

    mesh = plsc.VectorSubcoreMesh(core_axis_name="c", subcore_axis_name="s")

    @functools.partial(
        pl.kernel,
        mesh=mesh,
        compiler_params=pltpu.CompilerParams(use_tc_tiling_on_sc=False),
        out_type=jax.ShapeDtypeStruct((NW, nch, ROWS, D_MODEL), jnp.float32),
        scratch_types=[
            pltpu.VMEM((n_idx_rows, CHUNK), jnp.int32),
            pltpu.VMEM((ROWS, D_MODEL), jnp.float32),
            pltpu.SemaphoreType.DMA,
            pltpu.SemaphoreType.DMA,
        ],
    )
    def lookup(idx_hbm, table_hbm, out_hbm, idx_v, rows_v, gsem, wsem):
        wid = lax.axis_index("s") * NUM_CORES + lax.axis_index("c")
        pltpu.sync_copy(idx_hbm.at[wid], idx_v)

        def group(g, carry):
            copies = [
                pltpu.async_copy(
                    table_hbm.at[idx_v.at[g * K + j]],
                    rows_v.at[pl.ds(j * CHUNK, CHUNK)],
                    gsem,
                )
                for j in range(K)
            ]
            for c in copies:
                c.wait()

            def scale_row(r, carry2):
                for c in range(D_MODEL // LANES):
                    sl = pl.ds(c * LANES, LANES)
                    rows_v[r, sl] = rows_v[r, sl] * SCALE
                return carry2

            lax.fori_loop(0, ROWS, scale_row, 0, unroll=2)

            pltpu.async_copy(rows_v, out_hbm.at[wid, g], wsem).wait()
            return carry

        lax.fori_loop(0, nch, group, 0)

    return lookup


def kernel(input_ids, embeddings):
    b, h = input_ids.shape
    total = b * h
    idx = input_ids.reshape(NW, total // (NW * CHUNK), CHUNK).astype(jnp.int32)
    out = _make_lookup(total)(idx, embeddings)
    return out.reshape(b, h, D_MODEL)
